# baseline (speedup 1.0000x reference)
import functools

import jax
import jax.numpy as jnp
from jax import lax
from jax.experimental import pallas as pl
from jax.experimental.pallas import tpu as pltpu
from jax.experimental.pallas import tpu_sc as plsc

NUSERS = 60000
NITEMS = 40000
NNODES = NUSERS + NITEMS
DIM = 32
NEDGES = 1600000
BATCH = 4096

LANES = 128
EROWS = 12576
EPAD = EROWS * LANES
ROWS_PER_TILE = EROWS // 16
NCHUNKS = ROWS_PER_TILE
RING = 6
NGROUPS = NCHUNKS // RING

HALF = NNODES // 2
DUMMY = HALF
HALF_PAD = 50176
ZROWS = HALF_PAD // 16
WB_ROWS = 3128
WB_LAST = HALF - 15 * WB_ROWS

OUT_B = 3 * BATCH
OUT_ROWS = OUT_B // LANES
RPW = OUT_ROWS // 32
OUT_PER_W = RPW * LANES

_mesh = plsc.VectorSubcoreMesh(core_axis_name="c", subcore_axis_name="s")


@functools.partial(
    pl.kernel,
    out_type=jax.ShapeDtypeStruct((NNODES, DIM), jnp.float32),
    mesh=_mesh,
    compiler_params=pltpu.CompilerParams(
        use_tc_tiling_on_sc=False, needs_layout_passes=False),
    scratch_types=(
        [pltpu.VMEM_SHARED((HALF_PAD, DIM), jnp.float32)]
        + [pltpu.VMEM((1, 3, LANES), jnp.int32) for _ in range(RING)]
        + [pltpu.VMEM((LANES, DIM), jnp.float32) for _ in range(RING)]
        + [pltpu.SemaphoreType.DMA for _ in range(2 * RING)]
    ),
)
def _layer(rep_hbm, pck_hbm, out_hbm, acc_sh, *bufs):
    pck = bufs[0:RING]
    rows = bufs[RING:2 * RING]
    si = bufs[2 * RING:3 * RING]
    ss = bufs[3 * RING:4 * RING]
    cid = lax.axis_index("c")
    tid = lax.axis_index("s")
    lo = cid * HALF
    hi = lo + HALF

    def fire_load(j, b):
        pltpu.async_copy(pck_hbm.at[pl.ds(tid * ROWS_PER_TILE + j, 1)],
                         pck[b], si[b])

    def wait_load(j, b):
        pltpu.make_async_copy(pck_hbm.at[pl.ds(tid * ROWS_PER_TILE + j, 1)],
                              pck[b], si[b]).wait()

    def fire_gather(b):
        pltpu.async_copy(rep_hbm.at[pck[b].at[0, 0]], rows[b], si[b])

    def wait_gather(b):
        pltpu.make_async_copy(rep_hbm.at[pck[b].at[0, 0]], rows[b],
                              si[b]).wait()

    def fire_scatter(b):
        pltpu.async_copy(rows[b], acc_sh.at[pck[b].at[0, 1]], ss[b], add=True)

    def wait_scatter(b):
        pltpu.make_async_copy(rows[b], acc_sh.at[pck[b].at[0, 1]],
                              ss[b]).wait()

    def compute(b):
        def body(h, cc):
            sl = pl.ds(h * 16, 16)
            r16 = pck[b][0, 1, sl]
            owned = (r16 >= lo) & (r16 < hi)
            pck[b][0, 1, sl] = jnp.where(owned, r16 - lo, DUMMY)
            val16 = plsc.bitcast(pck[b][0, 2, sl], jnp.float32)
            for k in range(16):
                e = h * 16 + k
                s = val16[k]
                rows[b][e, pl.ds(0, 16)] = rows[b][e, pl.ds(0, 16)] * s
                rows[b][e, pl.ds(16, 16)] = rows[b][e, pl.ds(16, 16)] * s
            return cc

        lax.fori_loop(0, 8, body, 0)

    zero16 = jnp.zeros((16,), jnp.float32)

    def zbuf(e, c):
        rows[0][e, pl.ds(0, 16)] = zero16
        rows[0][e, pl.ds(16, 16)] = zero16
        return c

    lax.fori_loop(0, LANES, zbuf, 0)
    zb = tid * ZROWS
    zoff = 0
    while zoff < ZROWS:
        zn = min(LANES, ZROWS - zoff)
        pltpu.sync_copy(rows[0].at[pl.ds(0, zn)],
                        acc_sh.at[pl.ds(zb + zoff, zn)])
        zoff += zn
    plsc.subcore_barrier()

    for b in range(4):
        fire_load(b, b)
    for b in range(2):
        wait_load(b, b)
        fire_gather(b)

    def group(i, c):
        for t in range(RING):
            b = t
            b2 = (t + 2) % RING
            b4 = (t + 4) % RING

            if t < 2:
                @pl.when(i >= 1)
                def _():
                    wait_scatter(b4)

                fire_load(RING * i + t + 4, b4)
            else:
                wait_scatter(b4)

                @pl.when(i < NGROUPS - 1)
                def _():
                    fire_load(RING * i + t + 4, b4)

            if t < 4:
                wait_load(RING * i + t + 2, b2)
                fire_gather(b2)
            else:
                @pl.when(i < NGROUPS - 1)
                def _():
                    wait_load(RING * i + t + 2, b2)
                    fire_gather(b2)

            wait_gather(b)
            compute(b)
            fire_scatter(b)
        return c

    lax.fori_loop(0, NGROUPS, group, 0)
    wait_scatter(4)
    wait_scatter(5)
    plsc.subcore_barrier()

    wb = tid * WB_ROWS

    @pl.when(tid < 15)
    def _():
        pltpu.sync_copy(acc_sh.at[pl.ds(wb, WB_ROWS)],
                        out_hbm.at[pl.ds(lo + wb, WB_ROWS)])

    @pl.when(tid == 15)
    def _():
        pltpu.sync_copy(acc_sh.at[pl.ds(15 * WB_ROWS, WB_LAST)],
                        out_hbm.at[pl.ds(lo + 15 * WB_ROWS, WB_LAST)])


@functools.partial(
    pl.kernel,
    out_type=[
        jax.ShapeDtypeStruct((OUT_B, DIM), jnp.float32),
        jax.ShapeDtypeStruct((512,), jnp.float32),
    ],
    mesh=_mesh,
    compiler_params=pltpu.CompilerParams(use_tc_tiling_on_sc=False),
    scratch_types=[
        pltpu.VMEM((OUT_PER_W,), jnp.int32),
        pltpu.VMEM((OUT_PER_W, DIM), jnp.float32),
        pltpu.VMEM((OUT_PER_W, DIM), jnp.float32),
        pltpu.VMEM((OUT_PER_W, DIM), jnp.float32),
        pltpu.VMEM((OUT_PER_W, DIM), jnp.float32),
        pltpu.VMEM((16,), jnp.float32),
        pltpu.SemaphoreType.DMA,
    ],
)
def _final(r0h, r1h, r2h, r3h, idx_hbm, out_hbm, part_hbm,
           idx_v, b0, b1, b2, b3, part_v, sem):
    cid = lax.axis_index("c")
    tid = lax.axis_index("s")
    wid = tid * 2 + cid

    pltpu.sync_copy(idx_hbm.at[pl.ds(wid * OUT_PER_W, OUT_PER_W)], idx_v)
    cps = []
    for h, b in ((r0h, b0), (r1h, b1), (r2h, b2), (r3h, b3)):
        for g in range(RPW):
            cps.append(
                pltpu.async_copy(h.at[idx_v.at[pl.ds(g * LANES, LANES)]],
                                 b.at[pl.ds(g * LANES, LANES)], sem))
    for cp in cps:
        cp.wait()

    def cbody(e, p):
        for half in range(2):
            sl = pl.ds(half * 16, 16)
            x0 = b0[e, sl]
            p = p + x0 * x0
            b0[e, sl] = (x0 + b1[e, sl] + b2[e, sl] + b3[e, sl]) * 0.25
        return p

    p = lax.fori_loop(0, OUT_PER_W, cbody, jnp.zeros((16,), jnp.float32))
    part_v[pl.ds(0, 16)] = p

    pltpu.sync_copy(b0, out_hbm.at[pl.ds(wid * OUT_PER_W, OUT_PER_W)])
    pltpu.sync_copy(part_v, part_hbm.at[pl.ds(wid * 16, 16)])


def kernel(user_emb, item_emb, edge_row, edge_col, adj_vals,
           user_list, pos_items, neg_items):
    rep0 = jnp.concatenate([user_emb, item_emb], axis=0)
    pad = EPAD - NEDGES
    colp = jnp.concatenate(
        [edge_col.astype(jnp.int32), jnp.zeros((pad,), jnp.int32)]
    ).reshape(EROWS, LANES)
    rowp = jnp.concatenate(
        [edge_row.astype(jnp.int32), jnp.zeros((pad,), jnp.int32)]
    ).reshape(EROWS, LANES)
    valp = lax.bitcast_convert_type(
        jnp.concatenate([adj_vals, jnp.zeros((pad,), jnp.float32)]),
        jnp.int32,
    ).reshape(EROWS, LANES)
    pck = jnp.stack([colp, rowp, valp], axis=1)

    rep1 = _layer(rep0, pck)
    rep2 = _layer(rep1, pck)
    rep3 = _layer(rep2, pck)

    idx_all = jnp.concatenate([
        user_list.astype(jnp.int32),
        pos_items.astype(jnp.int32) + NUSERS,
        neg_items.astype(jnp.int32) + NUSERS,
    ])

    out, parts = _final(rep0, rep1, rep2, rep3, idx_all)
    reg = jnp.sum(parts) / BATCH
    return (out[:BATCH], out[BATCH:2 * BATCH], out[2 * BATCH:], reg)


# speedup vs baseline: 1.0103x; 1.0103x over previous
import functools

import jax
import jax.numpy as jnp
from jax import lax
from jax.experimental import pallas as pl
from jax.experimental.pallas import tpu as pltpu
from jax.experimental.pallas import tpu_sc as plsc

NUSERS = 60000
NITEMS = 40000
NNODES = NUSERS + NITEMS
DIM = 32
NEDGES = 1600000
BATCH = 4096

LANES = 128
EROWS = 12576
EPAD = EROWS * LANES
ROWS_PER_TILE = EROWS // 16
CROWS = 2
CHUNK_E = CROWS * LANES
NCHUNKS = ROWS_PER_TILE // CROWS
RING = 3
NGROUPS = NCHUNKS // RING

HALF = NNODES // 2
DUMMY = HALF
HALF_PAD = 50176
ZROWS = HALF_PAD // 16
WB_ROWS = 3128
WB_LAST = HALF - 15 * WB_ROWS

OUT_B = 3 * BATCH
OUT_ROWS = OUT_B // LANES
RPW = OUT_ROWS // 32
OUT_PER_W = RPW * LANES

_mesh = plsc.VectorSubcoreMesh(core_axis_name="c", subcore_axis_name="s")


@functools.partial(
    pl.kernel,
    out_type=jax.ShapeDtypeStruct((NNODES, DIM), jnp.float32),
    mesh=_mesh,
    compiler_params=pltpu.CompilerParams(
        use_tc_tiling_on_sc=False, needs_layout_passes=False),
    scratch_types=(
        [pltpu.VMEM_SHARED((HALF_PAD, DIM), jnp.float32)]
        + [pltpu.VMEM((3 * CHUNK_E,), jnp.int32) for _ in range(RING)]
        + [pltpu.VMEM((CHUNK_E, DIM), jnp.float32) for _ in range(RING)]
        + [pltpu.VMEM((CHUNK_E,), jnp.int32) for _ in range(RING)]
        + [pltpu.SemaphoreType.DMA for _ in range(3 * RING)]
    ),
)
def _layer(rep_hbm, pck_hbm, out_hbm, acc_sh, *bufs):
    pck = bufs[0:RING]
    rows = bufs[RING:2 * RING]
    ridx = bufs[2 * RING:3 * RING]
    sp = bufs[3 * RING:4 * RING]
    sg = bufs[4 * RING:5 * RING]
    ss = bufs[5 * RING:6 * RING]
    cid = lax.axis_index("c")
    tid = lax.axis_index("s")
    lo = cid * HALF
    hi = lo + HALF

    def fire_load(j, b):
        pltpu.async_copy(
            pck_hbm.at[pl.ds((tid * NCHUNKS + j) * 3 * CHUNK_E, 3 * CHUNK_E)],
            pck[b], sp[b])

    def wait_load(j, b):
        pltpu.make_async_copy(
            pck_hbm.at[pl.ds((tid * NCHUNKS + j) * 3 * CHUNK_E, 3 * CHUNK_E)],
            pck[b], sp[b]).wait()

    def fire_gather(b):
        pltpu.async_copy(rep_hbm.at[pck[b].at[pl.ds(0, CHUNK_E)]],
                         rows[b], sg[b])

    def wait_gather(b):
        pltpu.make_async_copy(rep_hbm.at[pck[b].at[pl.ds(0, CHUNK_E)]],
                              rows[b], sg[b]).wait()

    def fire_scatter(b):
        pltpu.async_copy(rows[b], acc_sh.at[ridx[b]], ss[b], add=True)

    def wait_scatter(b):
        pltpu.make_async_copy(rows[b], acc_sh.at[ridx[b]], ss[b]).wait()

    def compute(b):
        def body(h, cc):
            for g in range(CROWS):
                o = g * LANES + h * 16
                r16 = pck[b][pl.ds(CHUNK_E + o, 16)]
                owned = (r16 >= lo) & (r16 < hi)
                ridx[b][pl.ds(o, 16)] = jnp.where(owned, r16 - lo, DUMMY)
                val16 = plsc.bitcast(pck[b][pl.ds(2 * CHUNK_E + o, 16)],
                                     jnp.float32)
                for k in range(16):
                    e = g * LANES + h * 16 + k
                    s = val16[k]
                    rows[b][e, pl.ds(0, 16)] = rows[b][e, pl.ds(0, 16)] * s
                    rows[b][e, pl.ds(16, 16)] = rows[b][e, pl.ds(16, 16)] * s
            return cc

        lax.fori_loop(0, 8, body, 0)

    zero16 = jnp.zeros((16,), jnp.float32)

    def zbuf(e, c):
        rows[0][e, pl.ds(0, 16)] = zero16
        rows[0][e, pl.ds(16, 16)] = zero16
        return c

    lax.fori_loop(0, CHUNK_E, zbuf, 0)
    zb = tid * ZROWS
    zoff = 0
    while zoff < ZROWS:
        zn = min(CHUNK_E, ZROWS - zoff)
        pltpu.sync_copy(rows[0].at[pl.ds(0, zn)],
                        acc_sh.at[pl.ds(zb + zoff, zn)])
        zoff += zn
    plsc.subcore_barrier()

    fire_load(0, 0)
    fire_load(1, 1)
    wait_load(0, 0)
    fire_gather(0)

    def group(i, c):
        for t in range(RING):
            b = t
            b1 = (t + 1) % RING
            b2 = (t + 2) % RING

            if t < 2:
                @pl.when(i >= 1)
                def _():
                    wait_scatter(b1)
            else:
                wait_scatter(b1)

            if t == 0:
                fire_load(RING * i + t + 2, b2)
            else:
                @pl.when(i < NGROUPS - 1)
                def _():
                    fire_load(RING * i + t + 2, b2)

            if t < 2:
                wait_load(RING * i + t + 1, b1)
                fire_gather(b1)
            else:
                @pl.when(i < NGROUPS - 1)
                def _():
                    wait_load(RING * i + t + 1, b1)
                    fire_gather(b1)

            wait_gather(b)
            compute(b)
            fire_scatter(b)
        return c

    lax.fori_loop(0, NGROUPS, group, 0)
    wait_scatter(1)
    wait_scatter(2)
    plsc.subcore_barrier()

    wb = tid * WB_ROWS

    @pl.when(tid < 15)
    def _():
        pltpu.sync_copy(acc_sh.at[pl.ds(wb, WB_ROWS)],
                        out_hbm.at[pl.ds(lo + wb, WB_ROWS)])

    @pl.when(tid == 15)
    def _():
        pltpu.sync_copy(acc_sh.at[pl.ds(15 * WB_ROWS, WB_LAST)],
                        out_hbm.at[pl.ds(lo + 15 * WB_ROWS, WB_LAST)])


@functools.partial(
    pl.kernel,
    out_type=[
        jax.ShapeDtypeStruct((OUT_B, DIM), jnp.float32),
        jax.ShapeDtypeStruct((512,), jnp.float32),
    ],
    mesh=_mesh,
    compiler_params=pltpu.CompilerParams(use_tc_tiling_on_sc=False),
    scratch_types=[
        pltpu.VMEM((OUT_PER_W,), jnp.int32),
        pltpu.VMEM((OUT_PER_W, DIM), jnp.float32),
        pltpu.VMEM((OUT_PER_W, DIM), jnp.float32),
        pltpu.VMEM((OUT_PER_W, DIM), jnp.float32),
        pltpu.VMEM((OUT_PER_W, DIM), jnp.float32),
        pltpu.VMEM((16,), jnp.float32),
        pltpu.SemaphoreType.DMA,
    ],
)
def _final(r0h, r1h, r2h, r3h, idx_hbm, out_hbm, part_hbm,
           idx_v, b0, b1, b2, b3, part_v, sem):
    cid = lax.axis_index("c")
    tid = lax.axis_index("s")
    wid = tid * 2 + cid

    pltpu.sync_copy(idx_hbm.at[pl.ds(wid * OUT_PER_W, OUT_PER_W)], idx_v)
    cps = []
    for h, b in ((r0h, b0), (r1h, b1), (r2h, b2), (r3h, b3)):
        for g in range(RPW):
            cps.append(
                pltpu.async_copy(h.at[idx_v.at[pl.ds(g * LANES, LANES)]],
                                 b.at[pl.ds(g * LANES, LANES)], sem))
    for cp in cps:
        cp.wait()

    def cbody(e, p):
        for half in range(2):
            sl = pl.ds(half * 16, 16)
            x0 = b0[e, sl]
            p = p + x0 * x0
            b0[e, sl] = (x0 + b1[e, sl] + b2[e, sl] + b3[e, sl]) * 0.25
        return p

    p = lax.fori_loop(0, OUT_PER_W, cbody, jnp.zeros((16,), jnp.float32))
    part_v[pl.ds(0, 16)] = p

    pltpu.sync_copy(b0, out_hbm.at[pl.ds(wid * OUT_PER_W, OUT_PER_W)])
    pltpu.sync_copy(part_v, part_hbm.at[pl.ds(wid * 16, 16)])


def kernel(user_emb, item_emb, edge_row, edge_col, adj_vals,
           user_list, pos_items, neg_items):
    rep0 = jnp.concatenate([user_emb, item_emb], axis=0)
    pad = EPAD - NEDGES
    colp = jnp.concatenate(
        [edge_col.astype(jnp.int32), jnp.zeros((pad,), jnp.int32)]
    ).reshape(EROWS, LANES)
    rowp = jnp.concatenate(
        [edge_row.astype(jnp.int32), jnp.zeros((pad,), jnp.int32)]
    ).reshape(EROWS, LANES)
    valp = lax.bitcast_convert_type(
        jnp.concatenate([adj_vals, jnp.zeros((pad,), jnp.float32)]),
        jnp.int32,
    ).reshape(EROWS, LANES)
    pck = jnp.concatenate([
        colp.reshape(16, NCHUNKS, CROWS, LANES),
        rowp.reshape(16, NCHUNKS, CROWS, LANES),
        valp.reshape(16, NCHUNKS, CROWS, LANES),
    ], axis=2).reshape(16 * NCHUNKS * 3 * CHUNK_E)

    rep1 = _layer(rep0, pck)
    rep2 = _layer(rep1, pck)
    rep3 = _layer(rep2, pck)

    idx_all = jnp.concatenate([
        user_list.astype(jnp.int32),
        pos_items.astype(jnp.int32) + NUSERS,
        neg_items.astype(jnp.int32) + NUSERS,
    ])

    out, parts = _final(rep0, rep1, rep2, rep3, idx_all)
    reg = jnp.sum(parts) / BATCH
    return (out[:BATCH], out[BATCH:2 * BATCH], out[2 * BATCH:], reg)
